# baseline (speedup 1.0000x reference)
import functools

import jax
import jax.numpy as jnp
from jax import lax
from jax.experimental import pallas as pl
from jax.experimental.pallas import tpu as pltpu
from jax.experimental.pallas import tpu_sc as plsc

N = 10000
NTILE = 16
RPT = N // NTILE


def _embed_body(x_ref, w_ref, b_ref, o_ref):
    o_ref[...] = jnp.maximum(
        jnp.dot(x_ref[...], w_ref[...], preferred_element_type=jnp.float32)
        + b_ref[...],
        0.0,
    )


def _embed(x, W, b):
    n, d = x.shape
    h = W.shape[1]
    blk = 2000
    return pl.pallas_call(
        _embed_body,
        grid=(n // blk,),
        in_specs=[
            pl.BlockSpec((blk, d), lambda i: (i, 0)),
            pl.BlockSpec((d, h), lambda i: (0, 0)),
            pl.BlockSpec((1, h), lambda i: (0, 0)),
        ],
        out_specs=pl.BlockSpec((blk, h), lambda i: (i, 0)),
        out_shape=jax.ShapeDtypeStruct((n, h), jnp.float32),
    )(x, W, b.reshape(1, h))


def _spmm_half_kernel(src_ref, r_ref, c_ref, u_ref, acc, gbuf, cbuf, rbuf,
                      is0, is1, is2, is3, gs0, gs1, ss0, ss1):
    c = lax.axis_index("c")
    s = lax.axis_index("s")
    w = c * NTILE + s
    ntot = (r_ref.shape[0] // 128) // 32
    isems = (is0, is1, is2, is3)
    gsems = (gs0, gs1)
    ssems = (ss0, ss1)

    z16 = jnp.zeros((1, 16), jnp.float32)
    zbuf = gbuf.at[0]

    @pl.loop(0, 128)
    def _(i):
        for k8 in range(8):
            zbuf[pl.ds(i, 1), pl.ds(k8 * 16, 16)] = z16

    for j in range(5):
        pltpu.sync_copy(zbuf, acc.at[pl.ds(s * 640 + j * 128, 128)])
    plsc.subcore_barrier()

    def do_idx(k, p):
        off = (w * ntot + k) * 128
        pltpu.sync_copy(c_ref.at[pl.ds(off, 128)], cbuf.at[p])
        pltpu.sync_copy(r_ref.at[pl.ds(off, 128)], rbuf.at[p])

    def do_gather(p):
        pltpu.sync_copy(src_ref.at[cbuf.at[p]], gbuf.at[p])

    def sc_start(p):
        pltpu.async_copy(gbuf.at[p], acc.at[rbuf.at[p]], ssems[p], add=True)

    def sc_wait(p):
        pltpu.make_async_copy(gbuf.at[p], acc.at[rbuf.at[p]],
                              ssems[p]).wait()

    for p in (0, 1):
        do_idx(p, p)
        do_gather(p)
        sc_start(p)

    @pl.loop(1, ntot // 2)
    def _(i):
        for p in (0, 1):
            sc_wait(p)
            do_idx(2 * i + p, p)
            do_gather(p)
            sc_start(p)

    sc_wait(0)
    sc_wait(1)

    plsc.subcore_barrier()
    pltpu.sync_copy(acc.at[pl.ds(s * 640, 640)], u_ref.at[w])


def _spmm(rows, cols, src):
    mesh = plsc.VectorSubcoreMesh(core_axis_name="c", subcore_axis_name="s")
    k = pl.kernel(
        _spmm_half_kernel,
        out_type=jax.ShapeDtypeStruct((2 * NTILE, 640, 128), jnp.float32),
        mesh=mesh,
        scratch_types=[
            pltpu.VMEM_SHARED((NPAD, 128), jnp.float32),
            pltpu.VMEM((2, 128, 128), jnp.float32),
            pltpu.VMEM((4, 128), jnp.int32),
            pltpu.VMEM((4, 128), jnp.int32),
        ] + [pltpu.SemaphoreType.DMA] * 8,
    )
    u = k(src, rows, cols).reshape(2, NPAD, 128)
    return u[0] + u[1]


def _deg_kernel(r1_ref, c1_ref, r2_ref, c2_ref, out_ref,
                a0, a1, a2, a3, ones, ibuf, zbuf):
    c = lax.axis_index("c")
    s = lax.axis_index("s")
    accs = (a0, a1, a2, a3)

    z16 = jnp.zeros((16,), jnp.float32)
    o16 = jnp.full((16,), 1.0, jnp.float32)
    for k8 in range(8):
        ones[pl.ds(k8 * 16, 16)] = o16
    for k8 in range(40):
        zbuf[pl.ds(k8 * 16, 16)] = z16

    lo = s * 640
    for acc in accs:
        pltpu.sync_copy(zbuf, acc.at[pl.ds(lo, 640)])

    plsc.subcore_barrier()

    for m, idx_hbm in enumerate((r1_ref, c1_ref, r2_ref, c2_ref)):
        nch = idx_hbm.shape[0] // 128
        w = c * NTILE + s
        jlo = w * nch // 32
        jhi = (w + 1) * nch // 32

        @pl.loop(jlo, jhi)
        def _(j):
            pltpu.sync_copy(idx_hbm.at[pl.ds(j * 128, 128)], ibuf)
            pltpu.sync_copy(ones, accs[m].at[ibuf], add=True)

    plsc.subcore_barrier()

    for m in range(4):
        pltpu.sync_copy(accs[m].at[pl.ds(lo, 640)],
                        out_ref.at[c * 4 + m, 0, pl.ds(lo, 640)])


NPAD = 10240


def _degrees(r1, c1, r2, c2):
    mesh = plsc.VectorSubcoreMesh(core_axis_name="c", subcore_axis_name="s")
    k = pl.kernel(
        _deg_kernel,
        out_type=jax.ShapeDtypeStruct((8, 1, NPAD), jnp.float32),
        mesh=mesh,
        scratch_types=[
            pltpu.VMEM_SHARED((NPAD,), jnp.float32),
            pltpu.VMEM_SHARED((NPAD,), jnp.float32),
            pltpu.VMEM_SHARED((NPAD,), jnp.float32),
            pltpu.VMEM_SHARED((NPAD,), jnp.float32),
            pltpu.VMEM((128,), jnp.float32),
            pltpu.VMEM((128,), jnp.int32),
            pltpu.VMEM((640,), jnp.float32),
        ],
    )
    return k(r1, c1, r2, c2).reshape(2, 4, NPAD)


def _inv_body(d_ref, o_ref):
    d = jnp.sum(d_ref[...], axis=0)
    o_ref[...] = jnp.where(d > 0, jax.lax.rsqrt(d), 0.0)


def _inv(dpart):
    return pl.pallas_call(
        _inv_body,
        out_shape=jax.ShapeDtypeStruct((4, NPAD), jnp.float32),
    )(dpart)


def _scale_body(h_ref, d1_ref, d2_ref, o_ref):
    h = h_ref[...]
    o_ref[...] = jnp.concatenate([h * d1_ref[...], h * d2_ref[...]], axis=1)


def _scale_h(h, dc1, dc2, blk=2000):
    n = h.shape[0]
    return pl.pallas_call(
        _scale_body,
        grid=(n // blk,),
        in_specs=[
            pl.BlockSpec((blk, 128), lambda i: (i, 0)),
            pl.BlockSpec((blk, 1), lambda i: (i, 0)),
            pl.BlockSpec((blk, 1), lambda i: (i, 0)),
        ],
        out_specs=pl.BlockSpec((blk, 256), lambda i: (i, 0)),
        out_shape=jax.ShapeDtypeStruct((n, 256), jnp.float32),
    )(h, dc1.reshape(n, 1), dc2.reshape(n, 1))


def _mid_body(u1_ref, u2_ref, d1_ref, d2_ref, h1p_ref, sum_ref, sq_ref):
    i = pl.program_id(0)
    hp = jnp.concatenate(
        [u1_ref[...] * d1_ref[...], u2_ref[...] * d2_ref[...]], axis=1)
    h1p_ref[...] = hp

    @pl.when(i == 0)
    def _():
        sum_ref[...] = jnp.zeros_like(sum_ref)
        sq_ref[...] = jnp.zeros_like(sq_ref)

    sum_ref[...] += jnp.sum(hp, axis=0, keepdims=True)
    sq_ref[...] += jnp.sum(hp * hp, axis=0, keepdims=True)


def _mid(u1, u2, dr1, dr2, blk=2000):
    n = u1.shape[0]
    return pl.pallas_call(
        _mid_body,
        grid=(n // blk,),
        in_specs=[
            pl.BlockSpec((blk, 128), lambda i: (i, 0)),
            pl.BlockSpec((blk, 128), lambda i: (i, 0)),
            pl.BlockSpec((blk, 1), lambda i: (i, 0)),
            pl.BlockSpec((blk, 1), lambda i: (i, 0)),
        ],
        out_specs=[
            pl.BlockSpec((blk, 256), lambda i: (i, 0)),
            pl.BlockSpec((1, 256), lambda i: (0, 0)),
            pl.BlockSpec((1, 256), lambda i: (0, 0)),
        ],
        out_shape=[
            jax.ShapeDtypeStruct((n, 256), jnp.float32),
            jax.ShapeDtypeStruct((1, 256), jnp.float32),
            jax.ShapeDtypeStruct((1, 256), jnp.float32),
        ],
    )(u1, u2, dr1.reshape(n, 1), dr2.reshape(n, 1))


def _bn_body(hp_ref, mul_ref, add_ref, d1_ref, d2_ref, h1_ref, hs_ref):
    h1 = hp_ref[...] * mul_ref[...] + add_ref[...]
    h1_ref[...] = h1
    hs_ref[...] = jnp.concatenate([h1 * d1_ref[...], h1 * d2_ref[...]], axis=1)


def _bn_apply(h1p, mul, add, dc1, dc2, blk=2000):
    n = h1p.shape[0]
    return pl.pallas_call(
        _bn_body,
        grid=(n // blk,),
        in_specs=[
            pl.BlockSpec((blk, 256), lambda i: (i, 0)),
            pl.BlockSpec((1, 256), lambda i: (0, 0)),
            pl.BlockSpec((1, 256), lambda i: (0, 0)),
            pl.BlockSpec((blk, 1), lambda i: (i, 0)),
            pl.BlockSpec((blk, 1), lambda i: (i, 0)),
        ],
        out_specs=[
            pl.BlockSpec((blk, 256), lambda i: (i, 0)),
            pl.BlockSpec((blk, 512), lambda i: (i, 0)),
        ],
        out_shape=[
            jax.ShapeDtypeStruct((n, 256), jnp.float32),
            jax.ShapeDtypeStruct((n, 512), jnp.float32),
        ],
    )(h1p, mul, add, dc1.reshape(n, 1), dc2.reshape(n, 1))


def _final_body(h_ref, h1_ref, va_ref, vb_ref, vc_ref, vd_ref,
                d1_ref, d2_ref, w_ref, b_ref, o_ref):
    d1 = d1_ref[...]
    d2 = d2_ref[...]
    hj = jnp.concatenate([
        h_ref[...], h1_ref[...],
        va_ref[...] * d1, vb_ref[...] * d1,
        vc_ref[...] * d2, vd_ref[...] * d2,
    ], axis=1)
    acc = jnp.dot(hj, w_ref[...], preferred_element_type=jnp.float32)
    acc = acc + b_ref[...]
    m = jnp.max(acc, axis=1, keepdims=True)
    e = jnp.exp(acc - m)
    o_ref[...] = (acc - m) - jnp.log(jnp.sum(e, axis=1, keepdims=True))


def _final(h, h1, va, vb, vc, vd, dr1, dr2, W, b, blk=2000):
    n = h.shape[0]
    cdim = W.shape[1]
    return pl.pallas_call(
        _final_body,
        grid=(n // blk,),
        in_specs=[
            pl.BlockSpec((blk, 128), lambda i: (i, 0)),
            pl.BlockSpec((blk, 256), lambda i: (i, 0)),
            pl.BlockSpec((blk, 128), lambda i: (i, 0)),
            pl.BlockSpec((blk, 128), lambda i: (i, 0)),
            pl.BlockSpec((blk, 128), lambda i: (i, 0)),
            pl.BlockSpec((blk, 128), lambda i: (i, 0)),
            pl.BlockSpec((blk, 1), lambda i: (i, 0)),
            pl.BlockSpec((blk, 1), lambda i: (i, 0)),
            pl.BlockSpec((896, cdim), lambda i: (0, 0)),
            pl.BlockSpec((1, cdim), lambda i: (0, 0)),
        ],
        out_specs=pl.BlockSpec((blk, cdim), lambda i: (i, 0)),
        out_shape=jax.ShapeDtypeStruct((n, cdim), jnp.float32),
    )(h, h1, va, vb, vc, vd, dr1.reshape(n, 1), dr2.reshape(n, 1),
      W, b.reshape(1, cdim))


def kernel(x, edge_index, edge_index2, W_embed, b_embed, gamma0, beta0,
           W_final, b_final):
    n = x.shape[0]
    r1, c1 = edge_index[0], edge_index[1]
    r2, c2 = edge_index2[0], edge_index2[1]
    dpart = _degrees(r1, c1, r2, c2)
    inv = _inv(dpart)
    dr1, dc1, dr2, dc2 = inv[0, :n], inv[1, :n], inv[2, :n], inv[3, :n]

    def pad_e(e):
        epad = -e.shape[0] % 16384
        fill = N + (jnp.arange(epad, dtype=jnp.int32) % (NPAD - N))
        return jnp.concatenate([e, fill])

    r1p, c1p, r2p, c2p = pad_e(r1), pad_e(c1), pad_e(r2), pad_e(c2)

    def pad_n(y):
        return jnp.pad(y, ((0, NPAD - n), (0, 0)))

    h = _embed(x, W_embed, b_embed)
    hs = _scale_h(h, dc1, dc2)
    u1 = _spmm(r1p, c1p, pad_n(hs[:, :128]))[:n]
    u2 = _spmm(r2p, c2p, pad_n(hs[:, 128:]))[:n]
    h1p, ssum, ssq = _mid(u1, u2, dr1, dr2)
    mean = ssum / n
    var = ssq / n - mean * mean
    rsg = jax.lax.rsqrt(var + 1e-5) * gamma0
    h1, hsc = _bn_apply(h1p, rsg, beta0 - mean * rsg, dc1, dc2)
    v1a = _spmm(r1p, c1p, pad_n(hsc[:, 0:128]))[:n]
    v1b = _spmm(r1p, c1p, pad_n(hsc[:, 128:256]))[:n]
    v2a = _spmm(r2p, c2p, pad_n(hsc[:, 256:384]))[:n]
    v2b = _spmm(r2p, c2p, pad_n(hsc[:, 384:512]))[:n]
    return _final(h, h1, v1a, v1b, v2a, v2b, dr1, dr2, W_final, b_final)
